# baseline (speedup 1.0000x reference)
import functools

import jax
import jax.numpy as jnp
from jax import lax
from jax.experimental import pallas as pl
from jax.experimental.pallas import tpu as pltpu
from jax.experimental.pallas import tpu_sc as plsc

NTOKEN = 1000000
NINP = 32
BATCH = 16384
SEQ = 50

NC = 2
NS = 16
NW = NC * NS
TOT = BATCH * SEQ
PER_W = TOT // NW
CHUNK = 128
NCHUNK = PER_W // CHUNK


RING = 10
NGROUP = NCHUNK // RING


def _emb_body(idx_hbm, table_hbm, out_hbm, idx_v, rows_v, gsems, wsems):
    wid = lax.axis_index("s") * NC + lax.axis_index("c")
    pltpu.sync_copy(idx_hbm.at[wid], idx_v)

    base = wid * PER_W

    def group(i, carry):
        for b in range(RING):
            j = i * RING + b
            dst = out_hbm.at[pl.ds(base + j * CHUNK, CHUNK)]

            @pl.when(i > 0)
            def _():
                pltpu.make_async_copy(rows_v.at[b], dst, wsems[b]).wait()

            pltpu.make_async_copy(table_hbm.at[idx_v.at[j]], rows_v.at[b], gsems[b]).start()
        for b in range(RING):
            j = i * RING + b
            copy = pltpu.make_async_copy(table_hbm.at[idx_v.at[j]], rows_v.at[b], gsems[b])
            copy.wait()
            dst = out_hbm.at[pl.ds(base + j * CHUNK, CHUNK)]
            pltpu.make_async_copy(rows_v.at[b], dst, wsems[b]).start()
        return carry

    lax.fori_loop(0, NGROUP, group, 0)
    for b in range(RING):
        j = (NGROUP - 1) * RING + b
        dst = out_hbm.at[pl.ds(base + j * CHUNK, CHUNK)]
        pltpu.make_async_copy(rows_v.at[b], dst, wsems[b]).wait()


@jax.jit
def _emb(idx, table):
    mesh = plsc.VectorSubcoreMesh(core_axis_name="c", subcore_axis_name="s")
    k = pl.kernel(
        _emb_body,
        mesh=mesh,
        compiler_params=pltpu.CompilerParams(use_tc_tiling_on_sc=False),
        out_type=jax.ShapeDtypeStruct((TOT, NINP), jnp.float32),
        scratch_types=[
            pltpu.VMEM((NCHUNK, CHUNK), jnp.int32),
            pltpu.VMEM((RING, CHUNK, NINP), jnp.float32),
            [pltpu.SemaphoreType.DMA] * RING,
            [pltpu.SemaphoreType.DMA] * RING,
        ],
    )
    return k(idx, table)


def kernel(input_ids, embed_weight):
    idx = input_ids.reshape(-1).astype(jnp.int32).reshape(NW, NCHUNK, CHUNK)
    out = _emb(idx, embed_weight)
    return out.reshape(BATCH, SEQ, NINP)


# speedup vs baseline: 1.4655x; 1.4655x over previous
import functools

import jax
import jax.numpy as jnp
from jax import lax
from jax.experimental import pallas as pl
from jax.experimental.pallas import tpu as pltpu
from jax.experimental.pallas import tpu_sc as plsc

NTOKEN = 1000000
NINP = 32
BATCH = 16384
SEQ = 50

NC = 2
NS = 16
NW = NC * NS
BW = BATCH // NW
NBT = BW // 128
NG = NINP // 8


def _emb_body(idx_hbm, table_hbm, out_hbm, idx_v, a0, a1, b0, b1, gsems, wsems):
    wid = lax.axis_index("s") * NC + lax.axis_index("c")
    w0 = wid * NBT
    pltpu.sync_copy(idx_hbm.at[:, pl.ds(wid * BW, BW)], idx_v)

    A = (a0, a1)
    B = (b0, b1)

    def fire_gathers(l, p):
        for btl in range(NBT):
            src = table_hbm.at[idx_v.at[l, pl.ds(btl * 128, 128)]]
            pltpu.make_async_copy(src, A[p].at[pl.ds(btl * 128, 128)], gsems[p]).start()

    def wait_gathers(l, p):
        for btl in range(NBT):
            src = table_hbm.at[idx_v.at[l, pl.ds(btl * 128, 128)]]
            pltpu.make_async_copy(src, A[p].at[pl.ds(btl * 128, 128)], gsems[p]).wait()

    def fire_wb(l, p):
        for g in range(NG):
            pltpu.make_async_copy(B[p].at[g], out_hbm.at[l, g, pl.ds(w0, NBT)], wsems[p]).start()

    def wait_wb(l, p):
        for g in range(NG):
            pltpu.make_async_copy(B[p].at[g], out_hbm.at[l, g, pl.ds(w0, NBT)], wsems[p]).wait()

    def transpose(p):
        def fstep(f, carry):
            g = f // 8
            fi = f - g * 8
            fvec = jnp.full((16,), f, jnp.int32)
            for btl in range(NBT):
                for k in range(8):
                    bvec = jnp.arange(16, dtype=jnp.int32) + (btl * 128 + k * 16)
                    v = plsc.load_gather(A[p], [bvec, fvec])
                    B[p][g, btl, fi, pl.ds(k * 16, 16)] = v
            return carry

        lax.fori_loop(0, NINP, fstep, 0)

    fire_gathers(0, 0)

    def group(i, carry):
        l0 = 2 * i
        l1 = 2 * i + 1

        @pl.when(i > 0)
        def _():
            wait_wb(l1 - 2, 1)

        fire_gathers(l1, 1)
        wait_gathers(l0, 0)

        @pl.when(i > 0)
        def _():
            wait_wb(l0 - 2, 0)

        transpose(0)
        fire_wb(l0, 0)

        @pl.when(i < (SEQ // 2) - 1)
        def _():
            fire_gathers(l0 + 2, 0)

        wait_gathers(l1, 1)
        transpose(1)
        fire_wb(l1, 1)
        return carry

    lax.fori_loop(0, SEQ // 2, group, 0)
    wait_wb(SEQ - 2, 0)
    wait_wb(SEQ - 1, 1)


@jax.jit
def _emb(idxt, table):
    mesh = plsc.VectorSubcoreMesh(core_axis_name="c", subcore_axis_name="s")
    k = pl.kernel(
        _emb_body,
        mesh=mesh,
        compiler_params=pltpu.CompilerParams(
            use_tc_tiling_on_sc=False, needs_layout_passes=False
        ),
        out_type=jax.ShapeDtypeStruct((SEQ, NG, BATCH // 128, 8, 128), jnp.float32),
        scratch_types=[
            pltpu.VMEM((SEQ, BW), jnp.int32),
            pltpu.VMEM((BW, NINP), jnp.float32),
            pltpu.VMEM((BW, NINP), jnp.float32),
            pltpu.VMEM((NG, NBT, 8, 128), jnp.float32),
            pltpu.VMEM((NG, NBT, 8, 128), jnp.float32),
            [pltpu.SemaphoreType.DMA] * 2,
            [pltpu.SemaphoreType.DMA] * 2,
        ],
    )
    return k(idxt, table)


def kernel(input_ids, embed_weight):
    idxt = input_ids.T.astype(jnp.int32)
    o5 = _emb(idxt, embed_weight)
    return o5.transpose(2, 4, 0, 1, 3).reshape(BATCH, SEQ, NINP)


# speedup vs baseline: 1.9629x; 1.3394x over previous
import functools

import jax
import jax.numpy as jnp
from jax import lax
from jax.experimental import pallas as pl
from jax.experimental.pallas import tpu as pltpu
from jax.experimental.pallas import tpu_sc as plsc

NTOKEN = 1000000
NINP = 32
BATCH = 16384
SEQ = 50

NC = 2
NS = 16
NW = NC * NS
BW = BATCH // NW
NBT = BW // 128
NG = NINP // 8


def _emb_body(idx_hbm, table_hbm, out_hbm, idx_v, a0, a1, b0, b1, gsems, wsems):
    wid = lax.axis_index("s") * NC + lax.axis_index("c")
    w0 = wid * NBT
    pltpu.sync_copy(idx_hbm.at[:, pl.ds(wid * BW, BW)], idx_v)

    A = (a0, a1)
    B = (b0, b1)

    def fire_gathers(l, p):
        for btl in range(NBT):
            src = table_hbm.at[idx_v.at[l, pl.ds(btl * 128, 128)]]
            pltpu.make_async_copy(src, A[p].at[pl.ds(btl * 128, 128)], gsems[p]).start()

    def wait_gathers(l, p):
        for btl in range(NBT):
            src = table_hbm.at[idx_v.at[l, pl.ds(btl * 128, 128)]]
            pltpu.make_async_copy(src, A[p].at[pl.ds(btl * 128, 128)], gsems[p]).wait()

    def fire_wb(l, p):
        for g in range(NG):
            pltpu.make_async_copy(B[p].at[g], out_hbm.at[l, g, pl.ds(w0, NBT)], wsems[p]).start()

    def wait_wb(l, p):
        for g in range(NG):
            pltpu.make_async_copy(B[p].at[g], out_hbm.at[l, g, pl.ds(w0, NBT)], wsems[p]).wait()

    def transpose(p):
        @plsc.parallel_loop(0, NINP, unroll=4)
        def _(f):
            g = f // 8
            fi = f - g * 8
            fvec = jnp.full((16,), f, jnp.int32)
            for btl in range(NBT):
                for k in range(8):
                    bvec = jnp.arange(16, dtype=jnp.int32) + (btl * 128 + k * 16)
                    v = plsc.load_gather(A[p], [bvec, fvec])
                    B[p][g, btl, fi, pl.ds(k * 16, 16)] = v

    fire_gathers(0, 0)

    def group(i, carry):
        l0 = 2 * i
        l1 = 2 * i + 1

        @pl.when(i > 0)
        def _():
            wait_wb(l1 - 2, 1)

        fire_gathers(l1, 1)
        wait_gathers(l0, 0)

        @pl.when(i > 0)
        def _():
            wait_wb(l0 - 2, 0)

        transpose(0)
        fire_wb(l0, 0)

        @pl.when(i < (SEQ // 2) - 1)
        def _():
            fire_gathers(l0 + 2, 0)

        wait_gathers(l1, 1)
        transpose(1)
        fire_wb(l1, 1)
        return carry

    lax.fori_loop(0, SEQ // 2, group, 0)
    wait_wb(SEQ - 2, 0)
    wait_wb(SEQ - 1, 1)


@jax.jit
def _emb(idxt, table):
    mesh = plsc.VectorSubcoreMesh(core_axis_name="c", subcore_axis_name="s")
    k = pl.kernel(
        _emb_body,
        mesh=mesh,
        compiler_params=pltpu.CompilerParams(
            use_tc_tiling_on_sc=False, needs_layout_passes=False
        ),
        out_type=jax.ShapeDtypeStruct((SEQ, NG, BATCH // 128, 8, 128), jnp.float32),
        scratch_types=[
            pltpu.VMEM((SEQ, BW), jnp.int32),
            pltpu.VMEM((BW, NINP), jnp.float32),
            pltpu.VMEM((BW, NINP), jnp.float32),
            pltpu.VMEM((NG, NBT, 8, 128), jnp.float32),
            pltpu.VMEM((NG, NBT, 8, 128), jnp.float32),
            [pltpu.SemaphoreType.DMA] * 2,
            [pltpu.SemaphoreType.DMA] * 2,
        ],
    )
    return k(idxt, table)


def kernel(input_ids, embed_weight):
    idxt = input_ids.T.astype(jnp.int32)
    o5 = _emb(idxt, embed_weight)
    return o5.transpose(2, 4, 0, 1, 3).reshape(BATCH, SEQ, NINP)
